# Initial kernel scaffold; baseline (speedup 1.0000x reference)
import jax
import jax.numpy as jnp
from jax.experimental import pallas as pl


def kernel(input_, weight):
    raise NotImplementedError("write your pallas kernel here")


# speedup vs baseline: 1.7168x; 1.7168x over previous
import functools

import jax
import jax.numpy as jnp
from jax import lax
from jax.experimental import pallas as pl
from jax.experimental.pallas import tpu as pltpu
from jax.experimental.pallas import tpu_sc as plsc


@functools.lru_cache(maxsize=None)
def _make_kernel(B, L, D, V):
    info = plsc.get_sparse_core_info()
    NC, NS = info.num_cores, info.num_subcores
    NW = NC * NS
    bags_per_w = B // NW
    C = 8
    nchunks = bags_per_w // C
    IDX = C * L
    ND = D // 16
    inv_l = 1.0 / L

    mesh = plsc.VectorSubcoreMesh(core_axis_name="c", subcore_axis_name="s")

    @functools.partial(
        pl.kernel,
        mesh=mesh,
        compiler_params=pltpu.CompilerParams(use_tc_tiling_on_sc=False),
        out_type=jax.ShapeDtypeStruct((B, D), jnp.float32),
        scratch_types=[
            pltpu.VMEM((IDX,), jnp.int32),
            pltpu.VMEM((IDX, D), jnp.float32),
            pltpu.VMEM((bags_per_w, D), jnp.float32),
            pltpu.SemaphoreType.DMA,
        ],
    )
    def k(idx_hbm, table_hbm, out_hbm, idx_v, rows_v, out_v, sem):
        wid = lax.axis_index("s") * NC + lax.axis_index("c")
        bag_base = wid * bags_per_w

        def chunk_body(j, _):
            ib = (bag_base + j * C) * L
            pltpu.sync_copy(idx_hbm.at[pl.ds(ib, IDX)], idx_v)
            pltpu.async_copy(table_hbm.at[idx_v], rows_v, sem).wait()

            def bag_body(c, _):
                base = c * L
                for d in range(ND):
                    def lbody(l, acc):
                        return acc + rows_v[base + l, pl.ds(d * 16, 16)]

                    acc = lax.fori_loop(0, L, lbody,
                                        jnp.zeros((16,), jnp.float32))
                    out_v[j * C + c, pl.ds(d * 16, 16)] = acc * inv_l
                return 0

            lax.fori_loop(0, C, bag_body, 0)
            return 0

        lax.fori_loop(0, nchunks, chunk_body, 0)
        pltpu.sync_copy(out_v, out_hbm.at[pl.ds(bag_base, bags_per_w)])

    return k


def kernel(input_, weight):
    B, L = input_.shape
    V, D = weight.shape
    k = _make_kernel(B, L, D, V)
    return k(input_.reshape(-1), weight)


# speedup vs baseline: 2.7985x; 1.6301x over previous
import functools

import jax
import jax.numpy as jnp
from jax import lax
from jax.experimental import pallas as pl
from jax.experimental.pallas import tpu as pltpu
from jax.experimental.pallas import tpu_sc as plsc


@functools.lru_cache(maxsize=None)
def _make_kernel(B, L, D, V):
    info = plsc.get_sparse_core_info()
    NC, NS = info.num_cores, info.num_subcores
    NW = NC * NS
    bags_per_w = B // NW
    C = 8
    NB = 2
    nchunks = bags_per_w // C
    IDX = C * L
    ND = D // 16
    inv_l = 1.0 / L

    mesh = plsc.VectorSubcoreMesh(core_axis_name="c", subcore_axis_name="s")

    @functools.partial(
        pl.kernel,
        mesh=mesh,
        compiler_params=pltpu.CompilerParams(use_tc_tiling_on_sc=False),
        out_type=jax.ShapeDtypeStruct((B, D), jnp.float32),
        scratch_types=[
            pltpu.VMEM((bags_per_w * L,), jnp.int32),
            pltpu.VMEM((IDX, D), jnp.float32),
            pltpu.VMEM((IDX, D), jnp.float32),
            pltpu.VMEM((bags_per_w, D), jnp.float32),
            pltpu.SemaphoreType.DMA,
            pltpu.SemaphoreType.DMA,
        ],
    )
    def k(idx_hbm, table_hbm, out_hbm, idx_v, rows0, rows1, out_v, sem0,
          sem1):
        wid = lax.axis_index("s") * NC + lax.axis_index("c")
        bag_base = wid * bags_per_w
        pltpu.sync_copy(idx_hbm.at[pl.ds(bag_base * L, bags_per_w * L)],
                        idx_v)
        rows = (rows0, rows1)
        sems = (sem0, sem1)

        def gather_start(g, b):
            pltpu.async_copy(table_hbm.at[idx_v.at[pl.ds(g * IDX, IDX)]],
                             rows[b], sems[b])

        def gather_wait(b):
            pltpu.make_async_copy(
                table_hbm.at[idx_v.at[pl.ds(0, IDX)]], rows[b],
                sems[b]).wait()

        def compute(g, rv):
            def bag_body(c, _):
                base = c * L

                def lbody(l, accs):
                    r = base + l
                    return tuple(accs[d] + rv[r, pl.ds(d * 16, 16)]
                                 for d in range(ND))

                accs = lax.fori_loop(
                    0, L, lbody,
                    tuple(jnp.zeros((16,), jnp.float32) for _ in range(ND)),
                    unroll=10)
                row = g * C + c
                for d in range(ND):
                    out_v[row, pl.ds(d * 16, 16)] = accs[d] * inv_l
                return 0

            lax.fori_loop(0, C, bag_body, 0)

        for b in range(NB):
            gather_start(b, b)

        @pl.loop(0, nchunks, step=NB)
        def _(j):
            for b in range(NB):
                g = j + b
                gather_wait(b)
                compute(g, rows[b])

                @pl.when(g + NB < nchunks)
                def _():
                    gather_start(g + NB, b)

        pltpu.sync_copy(out_v, out_hbm.at[pl.ds(bag_base, bags_per_w)])

    return k


def kernel(input_, weight):
    B, L = input_.shape
    V, D = weight.shape
    k = _make_kernel(B, L, D, V)
    return k(input_.reshape(-1), weight)
